# Initial kernel scaffold; baseline (speedup 1.0000x reference)
import jax
import jax.numpy as jnp
from jax.experimental import pallas as pl


def kernel(mem, idx, val):
    raise NotImplementedError("write your pallas kernel here")


# speedup vs baseline: 1.1589x; 1.1589x over previous
import jax
import jax.numpy as jnp
from jax import lax
from jax.experimental import pallas as pl
from jax.experimental.pallas import tpu as pltpu
from jax.experimental.pallas import tpu_sc as plsc

M = 100000
D = 128
B = 16384

NC = 2
NS = 16
NW = NC * NS
ROWS_PER_W = M // NW
HALF = M // NC
CPY = 80
N_CPY = HALF // CPY
CH = 128
NCH_MAX = B // CH
NVEC = B // 16
BOARD = 3136


def _body(mem_hbm, idx_hbm, val_hbm, memnew_hbm, readback_hbm,
          idxbuf, jbuf, tbuf, board, rows2, cbuf,
          sem_in, sem_out, sem_g, sem_s):
    sc = lax.axis_index("c")
    tile = lax.axis_index("s")
    lo = sc * HALF + tile * ROWS_PER_W
    lane = lax.iota(jnp.int32, 16)

    base = sc * HALF

    def cpy_in(k, buf):
        return pltpu.make_async_copy(
            mem_hbm.at[pl.ds(base + k * CPY, CPY)], cbuf.at[buf], sem_in)

    def cpy_out(k, buf):
        return pltpu.make_async_copy(
            cbuf.at[buf], memnew_hbm.at[pl.ds(base + k * CPY, CPY)], sem_out)

    my_n = lax.div(N_CPY - tile + NS - 1, NS)
    cpy_in(tile, 0).start()

    def copy_body(i, _):
        k = tile + i * NS
        buf = lax.rem(i, 2)
        cpy_in(k, buf).wait()

        @pl.when(i >= 1)
        def _():
            cpy_out(k - NS, 1 - buf).wait()

        @pl.when(i < my_n - 1)
        def _():
            cpy_in(k + NS, 1 - buf).start()

        cpy_out(k, buf).start()
        return _

    lax.fori_loop(0, my_n, copy_body, None)

    pltpu.sync_copy(idx_hbm, idxbuf)

    def scan_body(i, count):
        t = idxbuf[pl.ds(i * 16, 16)]
        j = i * 16 + lane
        m = (t >= lo) & (t < lo + ROWS_PER_W)
        pos = count + plsc.cumsum(m.astype(jnp.int32)) - 1
        row = lax.shift_right_logical(pos, 7)
        col = lax.bitwise_and(pos, 127)
        plsc.store_scatter(jbuf, [row, col], j, mask=m)
        plsc.store_scatter(tbuf, [row, col], t, mask=m)
        return count + plsc.all_reduce_population_count(m)[0]

    count = lax.fori_loop(0, NVEC, scan_body, jnp.int32(0))
    ngrp = lax.div(count + 15, 16)
    nch = lax.div(count + (CH - 1), CH)

    def init_body(i, _):
        board[pl.ds(i * 16, 16)] = jnp.full((16,), -1, jnp.int32)
        return _

    lax.fori_loop(0, BOARD // 16, init_body, None)

    def cas_body(g, _):
        p = g * 16 + lane
        pm = p < count
        row = lax.shift_right_logical(p, 7)
        col = lax.bitwise_and(p, 127)
        tl = jnp.where(pm, plsc.load_gather(tbuf, [row, col]) - lo, 0)

        def cas_step(need):
            plsc.store_scatter(board, [tl], p, mask=need)
            cur = plsc.load_gather(board, [tl])
            return pm & (p > cur)

        need0 = pm & (p > plsc.load_gather(board, [tl]))
        lax.while_loop(jnp.any, cas_step, need0)
        return _

    lax.fori_loop(0, ngrp, cas_body, None)

    def fill_body(g, _):
        p = g * 16 + lane
        pm = p < count
        row = lax.shift_right_logical(p, 7)
        col = lax.bitwise_and(p, 127)
        tl = jnp.where(pm, plsc.load_gather(tbuf, [row, col]) - lo, 0)
        w = plsc.load_gather(board, [tl])
        wrow = lax.shift_right_logical(w, 7)
        wcol = lax.bitwise_and(w, 127)
        jw = plsc.load_gather(jbuf, [wrow, wcol])
        plsc.store_scatter(idxbuf, [p], jw, mask=pm)
        return _

    lax.fori_loop(0, ngrp, fill_body, None)

    @pl.when((count > 0) & (lax.rem(count, CH) != 0))
    def _():
        lrow = jnp.full((16,), lax.shift_right_logical(count - 1, 7), jnp.int32)
        lcol = jnp.full((16,), lax.bitwise_and(count - 1, 127), jnp.int32)
        jlast = plsc.load_gather(jbuf, [lrow, lcol])
        tlast = plsc.load_gather(tbuf, [lrow, lcol])
        jwl = plsc.load_gather(idxbuf, [jnp.full((16,), count - 1, jnp.int32)])
        for k in range(8):
            pos = count + k * 16 + lane
            pm2 = pos < nch * CH
            prow = lax.shift_right_logical(pos, 7)
            pcol = lax.bitwise_and(pos, 127)
            plsc.store_scatter(jbuf, [prow, pcol], jlast, mask=pm2)
            plsc.store_scatter(tbuf, [prow, pcol], tlast, mask=pm2)
            plsc.store_scatter(idxbuf, [pos], jwl, mask=pm2)

    pltpu.make_async_copy(
        cbuf.at[lax.rem(my_n - 1, 2)],
        memnew_hbm.at[pl.ds(base + (tile + (my_n - 1) * NS) * CPY, CPY)],
        sem_out).wait()
    plsc.subcore_barrier()

    def gat(c, buf):
        return pltpu.make_async_copy(
            val_hbm.at[idxbuf.at[pl.ds(c * CH, CH)]], rows2.at[buf], sem_g)

    def sca_mem(c, buf):
        return pltpu.make_async_copy(
            rows2.at[buf], memnew_hbm.at[tbuf.at[c]], sem_s)

    def sca_rb(c, buf):
        return pltpu.make_async_copy(
            rows2.at[buf], readback_hbm.at[jbuf.at[c]], sem_s)

    @pl.when(nch > 0)
    def _():
        gat(0, 0).start()

        def cd_body(c, _):
            buf = lax.rem(c, 2)
            gat(c, buf).wait()

            @pl.when(c >= 1)
            def _():
                sca_mem(c - 1, 1 - buf).wait()
                sca_rb(c - 1, 1 - buf).wait()

            @pl.when(c < nch - 1)
            def _():
                gat(c + 1, 1 - buf).start()

            sca_mem(c, buf).start()
            sca_rb(c, buf).start()
            return _

        lax.fori_loop(0, nch, cd_body, None)
        lbuf = lax.rem(nch - 1, 2)
        sca_mem(nch - 1, lbuf).wait()
        sca_rb(nch - 1, lbuf).wait()


@jax.jit
def _run(mem, idx, val):
    mesh = plsc.VectorSubcoreMesh(core_axis_name="c", subcore_axis_name="s")
    f = pl.kernel(
        _body,
        out_type=(
            jax.ShapeDtypeStruct((M, D), jnp.float32),
            jax.ShapeDtypeStruct((B, D), jnp.float32),
        ),
        mesh=mesh,
        compiler_params=pltpu.CompilerParams(needs_layout_passes=False),
        scratch_types=[
            pltpu.VMEM((B,), jnp.int32),
            pltpu.VMEM((NCH_MAX, CH), jnp.int32),
            pltpu.VMEM((NCH_MAX, CH), jnp.int32),
            pltpu.VMEM((BOARD,), jnp.int32),
            pltpu.VMEM((2, CH, D), jnp.float32),
            pltpu.VMEM((2, CPY, D), jnp.float32),
            pltpu.SemaphoreType.DMA,
            pltpu.SemaphoreType.DMA,
            pltpu.SemaphoreType.DMA,
            pltpu.SemaphoreType.DMA,
        ],
    )
    return f(mem, idx, val)


def kernel(mem, idx, val):
    return _run(mem, idx, val)
